# Initial kernel scaffold; baseline (speedup 1.0000x reference)
import jax
import jax.numpy as jnp
from jax.experimental import pallas as pl


def kernel(scalar, W_fc, emb_nan):
    raise NotImplementedError("write your pallas kernel here")


# speedup vs baseline: 7.1854x; 7.1854x over previous
import jax
import jax.numpy as jnp
from jax import lax
from jax.experimental import pallas as pl
from jax.experimental.pallas import tpu as pltpu
from jax.experimental.pallas import tpu_sc as plsc

L = 16
D = 128
B = 4096
SEQ = 50
NROWS = B * SEQ
NW = 32
B_W = B // NW
ROWS_W = NROWS // NW
NG = D // L
GROUPS = ((0, L), (16, L), (32, L), (48, 2))


def _body(s_hbm, w_hbm, e_hbm, out_hbm, s_v, w_v, e_v, buf0, buf1, sem0, sem1):
    wid = lax.axis_index("s") * 2 + lax.axis_index("c")
    base = wid * ROWS_W
    bbase = wid * B_W

    pltpu.sync_copy(s_hbm.at[pl.ds(base, ROWS_W)], s_v.at[pl.ds(0, ROWS_W)])
    pltpu.sync_copy(w_hbm, w_v)
    pltpu.sync_copy(e_hbm, e_v)

    wg = [w_v[pl.ds(g * L, L)] for g in range(NG)]
    e0g = [e_v[0, pl.ds(g * L, L)] for g in range(NG)]
    dg = [e_v[1, pl.ds(g * L, L)] - e0g[g] for g in range(NG)]

    bufs = (buf0, buf1)
    sems = (sem0, sem1)

    @pl.loop(0, B_W // 2)
    def _outer(i):
        for b in range(2):
            c = 2 * i + b
            buf = bufs[b]
            sem = sems[b]

            @pl.when(c >= 2)
            def _wait_prev():
                pltpu.make_async_copy(buf, out_hbm.at[bbase + c - 2], sem).wait()

            for off, cnt in GROUPS:
                sv = s_v[pl.ds(c * SEQ + off, L)]
                nanv = sv != sv
                cleanv = jnp.where(nanv, jnp.float32(0.0), sv)
                mfv = jnp.where(nanv, jnp.float32(1.0), jnp.float32(0.0))
                for j in range(cnt):
                    sb = jnp.full((L,), cleanv[j], jnp.float32)
                    mb = jnp.full((L,), mfv[j], jnp.float32)
                    for g in range(NG):
                        buf[off + j, pl.ds(g * L, L)] = sb * wg[g] + e0g[g] + mb * dg[g]

            pltpu.make_async_copy(buf, out_hbm.at[bbase + c], sem).start()

    pltpu.make_async_copy(buf0, out_hbm.at[bbase + B_W - 2], sem0).wait()
    pltpu.make_async_copy(buf1, out_hbm.at[bbase + B_W - 1], sem1).wait()


@jax.jit
def kernel(scalar, W_fc, emb_nan):
    s_flat = scalar.reshape(NROWS)
    w_flat = W_fc.reshape(D)

    run = pl.kernel(
        _body,
        out_type=jax.ShapeDtypeStruct((B, SEQ, D), jnp.float32),
        mesh=plsc.VectorSubcoreMesh(core_axis_name="c", subcore_axis_name="s"),
        scratch_types=[
            pltpu.VMEM((ROWS_W + L,), jnp.float32),
            pltpu.VMEM((D,), jnp.float32),
            pltpu.VMEM((2, D), jnp.float32),
            pltpu.VMEM((SEQ, D), jnp.float32),
            pltpu.VMEM((SEQ, D), jnp.float32),
            pltpu.SemaphoreType.DMA,
            pltpu.SemaphoreType.DMA,
        ],
    )
    return run(s_flat, w_flat, emb_nan)


# speedup vs baseline: 8.6641x; 1.2058x over previous
import jax
import jax.numpy as jnp
from jax import lax
from jax.experimental import pallas as pl
from jax.experimental.pallas import tpu as pltpu
from jax.experimental.pallas import tpu_sc as plsc

L = 16
D = 128
B = 4096
SEQ = 50
NW = 32
B_W = B // NW
NG = D // L
NBG = B_W // L


def _body(s_hbm, w_hbm, e_hbm, out_hbm, s_v, w_v, e_v, buf0, buf1, sem0, sem1):
    wid = lax.axis_index("s") * 2 + lax.axis_index("c")
    bbase = wid * B_W

    pltpu.sync_copy(s_hbm.at[:, pl.ds(bbase, B_W)], s_v)
    pltpu.sync_copy(w_hbm, w_v)
    pltpu.sync_copy(e_hbm, e_v)

    wg = [w_v[pl.ds(g * L, L)] for g in range(NG)]
    e0g = [e_v[0, pl.ds(g * L, L)] for g in range(NG)]
    dg = [e_v[1, pl.ds(g * L, L)] - e0g[g] for g in range(NG)]

    bufs = (buf0, buf1)
    sems = (sem0, sem1)

    @pl.loop(0, SEQ // 2)
    def _outer(i):
        for b in range(2):
            c = 2 * i + b
            buf = bufs[b]
            sem = sems[b]

            @pl.when(c >= 2)
            def _wait_prev():
                pltpu.make_async_copy(
                    buf, out_hbm.at[c - 2, pl.ds(bbase, B_W)], sem
                ).wait()

            for g in range(NBG):
                sv = s_v[c, pl.ds(g * L, L)]
                nanv = sv != sv
                cleanv = jnp.where(nanv, jnp.float32(0.0), sv)
                mfv = jnp.where(nanv, jnp.float32(1.0), jnp.float32(0.0))
                for j in range(L):
                    sb = jnp.full((L,), cleanv[j], jnp.float32)
                    mb = jnp.full((L,), mfv[j], jnp.float32)
                    r = g * L + j
                    for d in range(NG):
                        buf[r, pl.ds(d * L, L)] = sb * wg[d] + e0g[d] + mb * dg[d]

            pltpu.make_async_copy(
                buf, out_hbm.at[c, pl.ds(bbase, B_W)], sem
            ).start()

    pltpu.make_async_copy(
        buf0, out_hbm.at[SEQ - 2, pl.ds(bbase, B_W)], sem0
    ).wait()
    pltpu.make_async_copy(
        buf1, out_hbm.at[SEQ - 1, pl.ds(bbase, B_W)], sem1
    ).wait()


@jax.jit
def kernel(scalar, W_fc, emb_nan):
    s_t = jnp.transpose(scalar.reshape(B, SEQ), (1, 0))
    w_flat = W_fc.reshape(D)

    run = pl.kernel(
        _body,
        out_type=jax.ShapeDtypeStruct((SEQ, B, D), jnp.float32),
        mesh=plsc.VectorSubcoreMesh(core_axis_name="c", subcore_axis_name="s"),
        scratch_types=[
            pltpu.VMEM((SEQ, B_W), jnp.float32),
            pltpu.VMEM((D,), jnp.float32),
            pltpu.VMEM((2, D), jnp.float32),
            pltpu.VMEM((B_W, D), jnp.float32),
            pltpu.VMEM((B_W, D), jnp.float32),
            pltpu.SemaphoreType.DMA,
            pltpu.SemaphoreType.DMA,
        ],
    )
    out_t = run(s_t, w_flat, emb_nan)
    return jnp.transpose(out_t, (1, 0, 2))


# speedup vs baseline: 16.6515x; 1.9219x over previous
import jax
import jax.numpy as jnp
from jax import lax
from jax.experimental import pallas as pl
from jax.experimental.pallas import tpu as pltpu
from jax.experimental.pallas import tpu_sc as plsc

L = 16
D = 128
B = 4096
SEQ = 50
NW = 32
B_W = B // NW
NG = D // L
NBG = B_W // L


def _body(s_hbm, w_hbm, e_hbm, out_hbm, s_v, w_v, e_v, buf0, buf1, sem0, sem1):
    wid = lax.axis_index("s") * 2 + lax.axis_index("c")
    bbase = wid * B_W

    pltpu.sync_copy(s_hbm.at[:, pl.ds(bbase, B_W)], s_v)
    pltpu.sync_copy(w_hbm, w_v)
    pltpu.sync_copy(e_hbm, e_v)

    wg = [w_v[pl.ds(g * L, L)] for g in range(NG)]
    e0g = [e_v[0, pl.ds(g * L, L)] for g in range(NG)]
    dg = [e_v[1, pl.ds(g * L, L)] - e0g[g] for g in range(NG)]

    bufs = (buf0, buf1)
    sems = (sem0, sem1)

    @pl.loop(0, SEQ // 2)
    def _outer(i):
        for b in range(2):
            c = 2 * i + b
            buf = bufs[b]
            sem = sems[b]

            @pl.when(c >= 2)
            def _wait_prev():
                pltpu.make_async_copy(
                    buf, out_hbm.at[c - 2, pl.ds(bbase, B_W)], sem
                ).wait()

            @pl.loop(0, NBG)
            def _grp(g):
                sv = s_v[c, pl.ds(g * L, L)]
                nanv = sv != sv
                cleanv = jnp.where(nanv, jnp.float32(0.0), sv)
                mfv = jnp.where(nanv, jnp.float32(1.0), jnp.float32(0.0))
                for j in range(L):
                    sb = jnp.full((L,), cleanv[j], jnp.float32)
                    mb = jnp.full((L,), mfv[j], jnp.float32)
                    r = g * L + j
                    for d in range(NG):
                        buf[r, pl.ds(d * L, L)] = sb * wg[d] + e0g[d] + mb * dg[d]

            pltpu.make_async_copy(
                buf, out_hbm.at[c, pl.ds(bbase, B_W)], sem
            ).start()

    pltpu.make_async_copy(
        buf0, out_hbm.at[SEQ - 2, pl.ds(bbase, B_W)], sem0
    ).wait()
    pltpu.make_async_copy(
        buf1, out_hbm.at[SEQ - 1, pl.ds(bbase, B_W)], sem1
    ).wait()


@jax.jit
def kernel(scalar, W_fc, emb_nan):
    s_t = jnp.transpose(scalar.reshape(B, SEQ), (1, 0))
    w_flat = W_fc.reshape(D)

    run = pl.kernel(
        _body,
        out_type=jax.ShapeDtypeStruct((SEQ, B, D), jnp.float32),
        mesh=plsc.VectorSubcoreMesh(core_axis_name="c", subcore_axis_name="s"),
        scratch_types=[
            pltpu.VMEM((SEQ, B_W), jnp.float32),
            pltpu.VMEM((D,), jnp.float32),
            pltpu.VMEM((2, D), jnp.float32),
            pltpu.VMEM((B_W, D), jnp.float32),
            pltpu.VMEM((B_W, D), jnp.float32),
            pltpu.SemaphoreType.DMA,
            pltpu.SemaphoreType.DMA,
        ],
    )
    out_t = run(s_t, w_flat, emb_nan)
    return jnp.transpose(out_t, (1, 0, 2))
